# baseline (speedup 1.0000x reference)
import jax
import jax.numpy as jnp
from jax import lax
from jax.experimental import pallas as pl
from jax.experimental.pallas import tpu as pltpu

_L = 20
_V = 10
_B = 32768
_BLK = 16384


def _dot(a, b):
    return jnp.dot(a, b, preferred_element_type=jnp.float32)


def _col(row):
    return jnp.transpose(row, (1, 0))


def _dg0(w, x):
    return lax.dot_general(w, x, (((0,), (0,)), ((), ())),
                           preferred_element_type=jnp.float32)


def _body(xyt_ref, attt_ref, epst_ref, nanw_ref, nanb_ref, W1_ref, b1_ref,
          W2t_ref, b2_ref, G1_ref, gb1_ref, G2_ref, gb2_ref, P_ref,
          out_ref, acc_ref):
    i = pl.program_id(0)
    n = pl.num_programs(0)

    @pl.when(i == 0)
    def _init():
        acc_ref[...] = jnp.zeros_like(acc_ref)

    xyt = xyt_ref[...]
    attf = attt_ref[...].astype(jnp.float32)
    epst = epst_ref[...]

    W1f = nanw_ref[...].reshape(-1, 1) * W1_ref[...]
    b1f = _col(b1_ref[...]) + _dg0(W1_ref[...],
                                   nanb_ref[...].reshape(-1, 1))
    h1 = jnp.maximum(_dg0(W1f, xyt) + b1f, 0.0)
    mean = _dot(W2t_ref[:_V, :], h1) + _col(b2_ref[:, :_V])
    logv = _dot(W2t_ref[_V:, :], h1) + _col(b2_ref[:, _V:])
    s = epst * jnp.exp(0.5 * logv) + mean
    g = jnp.maximum(_dg0(G1_ref[...], s) + _col(gb1_ref[...]), 0.0)
    XY = _dg0(G2_ref[...], g) + _col(gb2_ref[...])

    d2 = (xyt - XY) ** 2
    cxen = (jnp.maximum(XY, 0.0) - XY * xyt
            + jnp.log1p(jnp.exp(-jnp.abs(XY))))
    fold2 = _dot(P_ref[...], d2)
    foldc = _dot(P_ref[...], cxen)
    l0_num = 0.5 * jnp.sum(attf * fold2)
    den = jnp.sum(attf)
    vae = -jnp.sum(attf * foldc) - 0.5 * jnp.sum(s * s - epst * epst - logv)

    upd = jnp.concatenate(
        [l0_num.reshape(1, 1), den.reshape(1, 1), vae.reshape(1, 1)], axis=1)
    acc_ref[...] += upd

    @pl.when(i == n - 1)
    def _finalize():
        acc = acc_ref[...]
        l0 = acc[0, 0] / jnp.maximum(acc[0, 1], 1.0)
        l_vae = -acc[0, 2] * (1.0 / _B)
        out_ref[...] = (l_vae + jnp.exp(l0)).reshape(1, 1)


def kernel(xy, att, eps, nan_w, nan_b, W1, b1, W2, b2, G1, gb1, G2, gb2):
    xyt = xy.T
    attt = att.view(jnp.int8).T
    epst = eps.T

    nblk = _B // _BLK
    slab = lambda h: pl.BlockSpec((h, _BLK), lambda i: (0, i))
    rep2 = lambda a, b: pl.BlockSpec((a, b), lambda i: (0, 0))

    out = pl.pallas_call(
        _body,
        grid=(nblk,),
        in_specs=[
            slab(2 * _L),
            slab(_L),
            slab(_V),
            rep2(1, 2 * _L),
            rep2(1, 2 * _L),
            rep2(2 * _L, 2 * _L),
            rep2(1, 2 * _L),
            rep2(2 * _V, 2 * _L),
            rep2(1, 2 * _V),
            rep2(_V, _V),
            rep2(1, _V),
            rep2(_V, 2 * _L),
            rep2(1, 2 * _L),
            rep2(_L, 2 * _L),
        ],
        out_specs=pl.BlockSpec((1, 1), lambda i: (0, 0)),
        out_shape=jax.ShapeDtypeStruct((1, 1), jnp.float32),
        scratch_shapes=[
            pltpu.VMEM((1, 3), jnp.float32),
        ],
    )(xyt, attt, epst,
      nan_w.reshape(1, -1), nan_b.reshape(1, -1), W1, b1.reshape(1, -1),
      W2.T, b2.reshape(1, -1), G1, gb1.reshape(1, -1), G2,
      gb2.reshape(1, -1),
      jnp.tile(jnp.eye(_L, dtype=jnp.float32), (1, 2)))
    return out[0, 0]


# speedup vs baseline: 1.0248x; 1.0248x over previous
import jax
import jax.numpy as jnp
from jax import lax
from jax.experimental import pallas as pl
from jax.experimental.pallas import tpu as pltpu

_L = 20
_V = 10
_B = 32768
_BLK = 32768


def _dot(a, b):
    return jnp.dot(a, b, preferred_element_type=jnp.float32)


def _col(row):
    return jnp.transpose(row, (1, 0))


def _dg0(w, x):
    return lax.dot_general(w, x, (((0,), (0,)), ((), ())),
                           preferred_element_type=jnp.float32)


def _body(xyt_ref, attt_ref, epst_ref, nanw_ref, nanb_ref, W1_ref, b1_ref,
          W2t_ref, b2_ref, G1_ref, gb1_ref, G2_ref, gb2_ref,
          out_ref, acc_ref):
    i = pl.program_id(0)
    n = pl.num_programs(0)

    @pl.when(i == 0)
    def _init():
        acc_ref[...] = jnp.zeros_like(acc_ref)

    xyt = xyt_ref[...]
    attf = attt_ref[...].astype(jnp.float32)
    epst = epst_ref[...]

    W1f = nanw_ref[...].reshape(-1, 1) * W1_ref[...]
    b1f = _col(b1_ref[...]) + _dg0(W1_ref[...],
                                   nanb_ref[...].reshape(-1, 1))
    h1 = jnp.maximum(_dg0(W1f, xyt) + b1f, 0.0)
    mean = _dot(W2t_ref[:_V, :], h1) + _col(b2_ref[:, :_V])
    logv = _dot(W2t_ref[_V:, :], h1) + _col(b2_ref[:, _V:])
    s = epst * jnp.exp(0.5 * logv) + mean
    g = jnp.maximum(_dg0(G1_ref[...], s) + _col(gb1_ref[...]), 0.0)
    XY = _dg0(G2_ref[...], g) + _col(gb2_ref[...])

    mask2 = jnp.concatenate([attf, attf], axis=0)
    d2 = (xyt - XY) ** 2
    l0_num = 0.5 * jnp.sum(d2 * mask2)
    den = 0.5 * jnp.sum(mask2)
    cxen = (jnp.maximum(XY, 0.0) - XY * xyt
            + jnp.log1p(jnp.exp(-jnp.abs(XY)))) * mask2
    vae = -jnp.sum(cxen) - 0.5 * jnp.sum(s * s - epst * epst - logv)

    upd = jnp.concatenate(
        [l0_num.reshape(1, 1), den.reshape(1, 1), vae.reshape(1, 1)], axis=1)
    acc_ref[...] += upd

    @pl.when(i == n - 1)
    def _finalize():
        acc = acc_ref[...]
        l0 = acc[0, 0] / jnp.maximum(acc[0, 1], 1.0)
        l_vae = -acc[0, 2] * (1.0 / _B)
        out_ref[...] = (l_vae + jnp.exp(l0)).reshape(1, 1)


def kernel(xy, att, eps, nan_w, nan_b, W1, b1, W2, b2, G1, gb1, G2, gb2):
    xyt = xy.T
    attt = att.view(jnp.int8).T
    epst = eps.T

    nblk = _B // _BLK
    slab = lambda h: pl.BlockSpec((h, _BLK), lambda i: (0, i))
    rep2 = lambda a, b: pl.BlockSpec((a, b), lambda i: (0, 0))

    out = pl.pallas_call(
        _body,
        grid=(nblk,),
        in_specs=[
            slab(2 * _L),
            slab(_L),
            slab(_V),
            rep2(1, 2 * _L),
            rep2(1, 2 * _L),
            rep2(2 * _L, 2 * _L),
            rep2(1, 2 * _L),
            rep2(2 * _V, 2 * _L),
            rep2(1, 2 * _V),
            rep2(_V, _V),
            rep2(1, _V),
            rep2(_V, 2 * _L),
            rep2(1, 2 * _L),
        ],
        out_specs=pl.BlockSpec((1, 1), lambda i: (0, 0)),
        out_shape=jax.ShapeDtypeStruct((1, 1), jnp.float32),
        scratch_shapes=[
            pltpu.VMEM((1, 3), jnp.float32),
        ],
    )(xyt, attt, epst,
      nan_w.reshape(1, -1), nan_b.reshape(1, -1), W1, b1.reshape(1, -1),
      W2.T, b2.reshape(1, -1), G1, gb1.reshape(1, -1), G2,
      gb2.reshape(1, -1))
    return out[0, 0]


# speedup vs baseline: 1.0561x; 1.0305x over previous
import jax
import jax.numpy as jnp
from jax import lax
from jax.experimental import pallas as pl
from jax.experimental.pallas import tpu as pltpu

_L = 20
_V = 10
_B = 32768
_BLK = 16384


def _dot(a, b):
    return jnp.dot(a, b, preferred_element_type=jnp.float32)


def _col(row):
    return jnp.transpose(row, (1, 0))


def _dg0(w, x):
    return lax.dot_general(w, x, (((0,), (0,)), ((), ())),
                           preferred_element_type=jnp.float32)


def _body(xyt_ref, attt_ref, epst_ref, nanw_ref, nanb_ref, W1_ref, b1_ref,
          W2t_ref, b2_ref, G1_ref, gb1_ref, G2_ref, gb2_ref,
          out_ref, acc_ref):
    i = pl.program_id(0)
    n = pl.num_programs(0)

    @pl.when(i == 0)
    def _init():
        acc_ref[...] = jnp.zeros_like(acc_ref)

    xyt = xyt_ref[...]
    attf = attt_ref[...].astype(jnp.float32)
    epst = epst_ref[...]

    W1f = nanw_ref[...].reshape(-1, 1) * W1_ref[...]
    b1f = _col(b1_ref[...]) + _dg0(W1_ref[...],
                                   nanb_ref[...].reshape(-1, 1))
    h1 = jnp.maximum(_dg0(W1f, xyt) + b1f, 0.0)
    mean = _dot(W2t_ref[:_V, :], h1) + _col(b2_ref[:, :_V])
    logv = _dot(W2t_ref[_V:, :], h1) + _col(b2_ref[:, _V:])
    s = epst * jnp.exp(0.5 * logv) + mean
    g = jnp.maximum(_dg0(G1_ref[...], s) + _col(gb1_ref[...]), 0.0)
    XY = _dg0(G2_ref[...], g) + _col(gb2_ref[...])

    mask2 = jnp.concatenate([attf, attf], axis=0)
    d2 = (xyt - XY) ** 2
    l0_num = 0.5 * jnp.sum(d2 * mask2)
    den = 0.5 * jnp.sum(mask2)
    cxen = (jnp.maximum(XY, 0.0) - XY * xyt
            + jnp.log1p(jnp.exp(-jnp.abs(XY)))) * mask2
    vae = -jnp.sum(cxen) - 0.5 * jnp.sum(s * s - epst * epst - logv)

    upd = jnp.concatenate(
        [l0_num.reshape(1, 1), den.reshape(1, 1), vae.reshape(1, 1)], axis=1)
    acc_ref[...] += upd

    @pl.when(i == n - 1)
    def _finalize():
        acc = acc_ref[...]
        l0 = acc[0, 0] / jnp.maximum(acc[0, 1], 1.0)
        l_vae = -acc[0, 2] * (1.0 / _B)
        out_ref[...] = (l_vae + jnp.exp(l0)).reshape(1, 1)


def kernel(xy, att, eps, nan_w, nan_b, W1, b1, W2, b2, G1, gb1, G2, gb2):
    xyt = xy.T
    attt = att.view(jnp.int8).T
    epst = eps.T

    nblk = _B // _BLK
    slab = lambda h: pl.BlockSpec((h, _BLK), lambda i: (0, i))
    rep2 = lambda a, b: pl.BlockSpec((a, b), lambda i: (0, 0))

    out = pl.pallas_call(
        _body,
        grid=(nblk,),
        in_specs=[
            slab(2 * _L),
            slab(_L),
            slab(_V),
            rep2(1, 2 * _L),
            rep2(1, 2 * _L),
            rep2(2 * _L, 2 * _L),
            rep2(1, 2 * _L),
            rep2(2 * _V, 2 * _L),
            rep2(1, 2 * _V),
            rep2(_V, _V),
            rep2(1, _V),
            rep2(_V, 2 * _L),
            rep2(1, 2 * _L),
        ],
        out_specs=pl.BlockSpec((1, 1), lambda i: (0, 0)),
        out_shape=jax.ShapeDtypeStruct((1, 1), jnp.float32),
        scratch_shapes=[
            pltpu.VMEM((1, 3), jnp.float32),
        ],
    )(xyt, attt, epst,
      nan_w.reshape(1, -1), nan_b.reshape(1, -1), W1, b1.reshape(1, -1),
      W2.T, b2.reshape(1, -1), G1, gb1.reshape(1, -1), G2,
      gb2.reshape(1, -1))
    return out[0, 0]
